# Initial kernel scaffold; baseline (speedup 1.0000x reference)
import jax
import jax.numpy as jnp
from jax.experimental import pallas as pl


def kernel(x, edge_index, batch, W0, b0, Wc1, bc1, Wc2, bc2, Wc3, bc3, Wl1, bl1, Wl2, bl2):
    raise NotImplementedError("write your pallas kernel here")


# speedup vs baseline: 18.0722x; 18.0722x over previous
import functools

import jax
import jax.numpy as jnp
from jax import lax
from jax.experimental import pallas as pl
from jax.experimental.pallas import tpu as pltpu
from jax.experimental.pallas import tpu_sc as plsc

N = 10000
E = 320000
D = 128
G = 64

NC = 2
NS = 16
NW = NC * NS
C = 128
EPT = -(-E // NW)
NCHUNK = -(-EPT // C)
EPT_PAD = NCHUNK * C
E_PAD = EPT_PAD * NW
NDUMMY = 112
NPAD = N + NDUMMY
ROWS_PT = NPAD // NS

_mesh = plsc.VectorSubcoreMesh(core_axis_name="c", subcore_axis_name="s")


@functools.partial(
    pl.kernel,
    out_type=jax.ShapeDtypeStruct((NC, NPAD, D), jnp.float32),
    mesh=_mesh,
    scratch_types=[
        pltpu.VMEM((NCHUNK, C), jnp.int32),
        pltpu.VMEM((C, D), jnp.float32),
        pltpu.VMEM_SHARED((NPAD, D), jnp.float32),
    ],
)
def _sc_degree(dst_hbm, ones_hbm, zeros_hbm, out_hbm, dst_v, ones_v, deg_sh):
    cid = lax.axis_index("c")
    sid = lax.axis_index("s")
    tid = cid * NS + sid
    r0 = sid * ROWS_PT
    pltpu.sync_copy(zeros_hbm.at[pl.ds(r0, ROWS_PT)], deg_sh.at[pl.ds(r0, ROWS_PT)])
    pltpu.sync_copy(ones_hbm, ones_v)
    pltpu.sync_copy(dst_hbm.at[tid], dst_v)
    plsc.subcore_barrier()

    def body(j, carry):
        pltpu.sync_copy(ones_v, deg_sh.at[dst_v.at[j]], add=True)
        return carry

    lax.fori_loop(0, NCHUNK, body, 0)
    plsc.subcore_barrier()
    pltpu.sync_copy(deg_sh.at[pl.ds(r0, ROWS_PT)], out_hbm.at[cid, pl.ds(r0, ROWS_PT)])


@functools.partial(
    pl.kernel,
    out_type=jax.ShapeDtypeStruct((NC, NPAD, D), jnp.float32),
    mesh=_mesh,
    scratch_types=[
        pltpu.VMEM((NCHUNK, C), jnp.int32),
        pltpu.VMEM((NCHUNK, C), jnp.int32),
        pltpu.VMEM((C, D), jnp.float32),
        pltpu.VMEM_SHARED((NPAD, D), jnp.float32),
        pltpu.SemaphoreType.DMA,
    ],
)
def _sc_edge_agg(hn_hbm, zeros_hbm, src_hbm, dst_hbm, out_hbm,
                 src_v, dst_v, rows_v, acc_sh, sem):
    cid = lax.axis_index("c")
    sid = lax.axis_index("s")
    tid = cid * NS + sid
    r0 = sid * ROWS_PT

    @pl.when(cid == 0)
    def _():
        pltpu.sync_copy(hn_hbm.at[pl.ds(r0, ROWS_PT)], acc_sh.at[pl.ds(r0, ROWS_PT)])

    @pl.when(cid != 0)
    def _():
        pltpu.sync_copy(zeros_hbm.at[pl.ds(r0, ROWS_PT)], acc_sh.at[pl.ds(r0, ROWS_PT)])

    pltpu.sync_copy(src_hbm.at[tid], src_v)
    pltpu.sync_copy(dst_hbm.at[tid], dst_v)
    plsc.subcore_barrier()

    def body(j, carry):
        pltpu.async_copy(hn_hbm.at[src_v.at[j]], rows_v, sem).wait()
        pltpu.sync_copy(rows_v, acc_sh.at[dst_v.at[j]], add=True)
        return carry

    lax.fori_loop(0, NCHUNK, body, 0)
    plsc.subcore_barrier()
    pltpu.sync_copy(acc_sh.at[pl.ds(r0, ROWS_PT)], out_hbm.at[cid, pl.ds(r0, ROWS_PT)])


def _dis_col(dsum):
    deg = dsum[0, :, 0:1] + dsum[1, :, 0:1] + 1.0
    dis = lax.rsqrt(jnp.maximum(deg, 1e-12))
    return dis[:N, :]


def _tc_in_body(x_ref, w0_ref, b0_ref, wc1_ref, dsum_ref, out_ref):
    disc = _dis_col(dsum_ref[...])
    h0 = jnp.maximum(
        lax.dot_general(x_ref[...], w0_ref[...], (((1,), (1,)), ((), ())))
        + b0_ref[...], 0.0)
    t = lax.dot_general(h0, wc1_ref[...], (((1,), (1,)), ((), ())))
    out_ref[0:N, :] = t * disc
    out_ref[N:NPAD, :] = jnp.zeros((NDUMMY, D), jnp.float32)


_tc_in = pl.pallas_call(
    _tc_in_body,
    out_shape=jax.ShapeDtypeStruct((NPAD, D), jnp.float32),
)


def _tc_mid_body(agg_ref, dsum_ref, b_ref, w_ref, out_ref):
    disc = _dis_col(dsum_ref[...])
    s = agg_ref[0, 0:N, :] + agg_ref[1, 0:N, :]
    y = jnp.maximum(s * disc + b_ref[...], 0.0)
    t = lax.dot_general(y, w_ref[...], (((1,), (1,)), ((), ())))
    out_ref[0:N, :] = t * disc
    out_ref[N:NPAD, :] = jnp.zeros((NDUMMY, D), jnp.float32)


_tc_mid = pl.pallas_call(
    _tc_mid_body,
    out_shape=jax.ShapeDtypeStruct((NPAD, D), jnp.float32),
)


def _tc_out_body(agg_ref, dsum_ref, b_ref, batch_ref, wl1_ref, bl1_ref,
                 wl2_ref, bl2_ref, out_ref):
    disc = _dis_col(dsum_ref[...])
    s = agg_ref[0, 0:N, :] + agg_ref[1, 0:N, :]
    y = jnp.maximum(s * disc + b_ref[...], 0.0)
    seg = lax.broadcasted_iota(jnp.int32, (1, G), 1)
    onehot = (batch_ref[...] == seg).astype(jnp.float32)
    g = lax.dot_general(onehot, y, (((0,), (0,)), ((), ())))
    g2 = jnp.maximum(
        lax.dot_general(g, wl1_ref[...], (((1,), (1,)), ((), ())))
        + bl1_ref[...], 0.0)
    o = lax.dot_general(g2, wl2_ref[...], (((1,), (1,)), ((), ()))) + bl2_ref[...]
    out_ref[...] = o


_tc_out = pl.pallas_call(
    _tc_out_body,
    out_shape=jax.ShapeDtypeStruct((G, 8), jnp.float32),
)


def kernel(x, edge_index, batch, W0, b0, Wc1, bc1, Wc2, bc2, Wc3, bc3,
           Wl1, bl1, Wl2, bl2):
    src = edge_index[0]
    dst = edge_index[1]
    pad_n = E_PAD - E
    padv = (N + (jnp.arange(pad_n, dtype=jnp.int32) % NDUMMY)).astype(jnp.int32)
    src_r = jnp.concatenate([src, padv]).reshape(NW, NCHUNK, C)
    dst_r = jnp.concatenate([dst, padv]).reshape(NW, NCHUNK, C)
    onesD = jnp.ones((C, D), jnp.float32)
    zerosD = jnp.zeros((NPAD, D), jnp.float32)

    dsum = _sc_degree(dst_r, onesD, zerosD)

    hn1 = _tc_in(x, W0, b0.reshape(1, D), Wc1, dsum)
    agg1 = _sc_edge_agg(hn1, zerosD, src_r, dst_r)
    hn2 = _tc_mid(agg1, dsum, bc1.reshape(1, D), Wc2)
    agg2 = _sc_edge_agg(hn2, zerosD, src_r, dst_r)
    hn3 = _tc_mid(agg2, dsum, bc2.reshape(1, D), Wc3)
    agg3 = _sc_edge_agg(hn3, zerosD, src_r, dst_r)

    wl2_pad = jnp.concatenate([Wl2, jnp.zeros((7, D), jnp.float32)], axis=0)
    bl2_pad = jnp.concatenate([bl2, jnp.zeros((7,), jnp.float32)]).reshape(1, 8)
    o = _tc_out(agg3, dsum, bc3.reshape(1, D), batch.reshape(N, 1),
                Wl1, bl1.reshape(1, D), wl2_pad, bl2_pad)
    return o[:, 0]


# speedup vs baseline: 25.1572x; 1.3920x over previous
import functools

import jax
import jax.numpy as jnp
from jax import lax
from jax.experimental import pallas as pl
from jax.experimental.pallas import tpu as pltpu
from jax.experimental.pallas import tpu_sc as plsc

N = 10000
E = 320000
D = 128
G = 64

NC = 2
NS = 16
NW = NC * NS
C = 128
EPT = -(-E // NW)
NPHASE = 2
NCHUNK = 4 * (-(-EPT // (4 * C)))
CH_P = NCHUNK // NPHASE
NHALF_P = CH_P // 2
EPT_PAD = NCHUNK * C
E_PAD = EPT_PAD * NW
NDUMMY = 112
NPAD = N + NDUMMY
ROWS_PT = NPAD // NS

_mesh = plsc.VectorSubcoreMesh(core_axis_name="c", subcore_axis_name="s")


@functools.partial(
    pl.kernel,
    out_type=jax.ShapeDtypeStruct((NC, NPAD, D), jnp.float32),
    mesh=_mesh,
    scratch_types=[
        pltpu.VMEM((NCHUNK, C), jnp.int32),
        pltpu.VMEM((C, D), jnp.float32),
        pltpu.VMEM_SHARED((NPAD, D), jnp.float32),
    ],
)
def _sc_degree(dst_hbm, ones_hbm, zeros_hbm, out_hbm, dst_v, ones_v, deg_sh):
    cid = lax.axis_index("c")
    sid = lax.axis_index("s")
    tid = cid * NS + sid
    r0 = sid * ROWS_PT
    pltpu.sync_copy(zeros_hbm.at[pl.ds(r0, ROWS_PT)], deg_sh.at[pl.ds(r0, ROWS_PT)])
    pltpu.sync_copy(ones_hbm, ones_v)
    pltpu.sync_copy(dst_hbm.at[tid], dst_v)
    plsc.subcore_barrier()

    def body(j, carry):
        pltpu.sync_copy(ones_v, deg_sh.at[dst_v.at[j]], add=True)
        return carry

    lax.fori_loop(0, NCHUNK, body, 0)
    plsc.subcore_barrier()
    pltpu.sync_copy(deg_sh.at[pl.ds(r0, ROWS_PT)], out_hbm.at[cid, pl.ds(r0, ROWS_PT)])


@functools.partial(
    pl.kernel,
    out_type=jax.ShapeDtypeStruct((NC, NPAD, D), jnp.float32),
    mesh=_mesh,
    scratch_types=[
        pltpu.VMEM((CH_P, C), jnp.int32),
        pltpu.VMEM((CH_P, C), jnp.int32),
        pltpu.VMEM((C, D), jnp.float32),
        pltpu.VMEM((C, D), jnp.float32),
        pltpu.VMEM_SHARED((NPAD, D), jnp.float32),
        pltpu.SemaphoreType.DMA,
        pltpu.SemaphoreType.DMA,
    ],
)
def _sc_edge_agg(hn_hbm, zeros_hbm, src_hbm, dst_hbm, out_hbm,
                 src_v, dst_v, rows_a, rows_b, acc_sh, sem_a, sem_b):
    cid = lax.axis_index("c")
    sid = lax.axis_index("s")
    tid = cid * NS + sid
    r0 = sid * ROWS_PT

    @pl.when(cid == 0)
    def _():
        pltpu.sync_copy(hn_hbm.at[pl.ds(r0, ROWS_PT)], acc_sh.at[pl.ds(r0, ROWS_PT)])

    @pl.when(cid != 0)
    def _():
        pltpu.sync_copy(zeros_hbm.at[pl.ds(r0, ROWS_PT)], acc_sh.at[pl.ds(r0, ROWS_PT)])

    plsc.subcore_barrier()

    def body(i, carry):
        j0 = 2 * i
        j1 = j0 + 1
        pltpu.async_copy(hn_hbm.at[src_v.at[j1]], rows_b, sem_b)
        pltpu.make_async_copy(hn_hbm.at[src_v.at[j0]], rows_a, sem_a).wait()
        pltpu.sync_copy(rows_a, acc_sh.at[dst_v.at[j0]], add=True)

        @pl.when(i < NHALF_P - 1)
        def _():
            pltpu.async_copy(hn_hbm.at[src_v.at[j0 + 2]], rows_a, sem_a)

        pltpu.make_async_copy(hn_hbm.at[src_v.at[j1]], rows_b, sem_b).wait()
        pltpu.sync_copy(rows_b, acc_sh.at[dst_v.at[j1]], add=True)
        return carry

    for p in range(NPHASE):
        pltpu.sync_copy(src_hbm.at[tid, pl.ds(p * CH_P, CH_P)], src_v)
        pltpu.sync_copy(dst_hbm.at[tid, pl.ds(p * CH_P, CH_P)], dst_v)
        pltpu.async_copy(hn_hbm.at[src_v.at[0]], rows_a, sem_a)
        lax.fori_loop(0, NHALF_P, body, 0)
    plsc.subcore_barrier()
    pltpu.sync_copy(acc_sh.at[pl.ds(r0, ROWS_PT)], out_hbm.at[cid, pl.ds(r0, ROWS_PT)])


def _dis_col(dsum):
    deg = dsum[0, :, 0:1] + dsum[1, :, 0:1] + 1.0
    dis = lax.rsqrt(jnp.maximum(deg, 1e-12))
    return dis[:N, :]


def _tc_h0_body(x_ref, w0_ref, b0_ref, out_ref):
    out_ref[...] = jnp.maximum(
        lax.dot_general(x_ref[...], w0_ref[...], (((1,), (1,)), ((), ())))
        + b0_ref[...], 0.0)


_tc_h0 = pl.pallas_call(
    _tc_h0_body,
    out_shape=jax.ShapeDtypeStruct((N, D), jnp.float32),
)


def _tc_in_body(h0_ref, wc1_ref, dsum_ref, out_ref):
    disc = _dis_col(dsum_ref[...])
    t = lax.dot_general(h0_ref[...], wc1_ref[...], (((1,), (1,)), ((), ())))
    out_ref[0:N, :] = t * disc
    out_ref[N:NPAD, :] = jnp.zeros((NDUMMY, D), jnp.float32)


_tc_in = pl.pallas_call(
    _tc_in_body,
    out_shape=jax.ShapeDtypeStruct((NPAD, D), jnp.float32),
)


def _tc_mid_body(agg_ref, dsum_ref, b_ref, w_ref, out_ref):
    disc = _dis_col(dsum_ref[...])
    s = agg_ref[0, 0:N, :] + agg_ref[1, 0:N, :]
    y = jnp.maximum(s * disc + b_ref[...], 0.0)
    t = lax.dot_general(y, w_ref[...], (((1,), (1,)), ((), ())))
    out_ref[0:N, :] = t * disc
    out_ref[N:NPAD, :] = jnp.zeros((NDUMMY, D), jnp.float32)


_tc_mid = pl.pallas_call(
    _tc_mid_body,
    out_shape=jax.ShapeDtypeStruct((NPAD, D), jnp.float32),
)


def _tc_out_body(agg_ref, dsum_ref, b_ref, batch_ref, wl1_ref, bl1_ref,
                 wl2_ref, bl2_ref, out_ref):
    disc = _dis_col(dsum_ref[...])
    s = agg_ref[0, 0:N, :] + agg_ref[1, 0:N, :]
    y = jnp.maximum(s * disc + b_ref[...], 0.0)
    seg = lax.broadcasted_iota(jnp.int32, (1, G), 1)
    onehot = (batch_ref[...] == seg).astype(jnp.float32)
    g = lax.dot_general(onehot, y, (((0,), (0,)), ((), ())))
    g2 = jnp.maximum(
        lax.dot_general(g, wl1_ref[...], (((1,), (1,)), ((), ())))
        + bl1_ref[...], 0.0)
    o = lax.dot_general(g2, wl2_ref[...], (((1,), (1,)), ((), ()))) + bl2_ref[...]
    out_ref[...] = o


_tc_out = pl.pallas_call(
    _tc_out_body,
    out_shape=jax.ShapeDtypeStruct((G, 8), jnp.float32),
)


def kernel(x, edge_index, batch, W0, b0, Wc1, bc1, Wc2, bc2, Wc3, bc3,
           Wl1, bl1, Wl2, bl2):
    src = edge_index[0]
    dst = edge_index[1]
    pad_n = E_PAD - E
    padv = (N + (jnp.arange(pad_n, dtype=jnp.int32) % NDUMMY)).astype(jnp.int32)
    src_r = jnp.concatenate([src, padv]).reshape(NW, NCHUNK, C)
    dst_r = jnp.concatenate([dst, padv]).reshape(NW, NCHUNK, C)
    onesD = jnp.ones((C, D), jnp.float32)
    zerosD = jnp.zeros((NPAD, D), jnp.float32)

    dsum = _sc_degree(dst_r, onesD, zerosD)

    h0 = _tc_h0(x, W0, b0.reshape(1, D))
    hn1 = _tc_in(h0, Wc1, dsum)
    agg1 = _sc_edge_agg(hn1, zerosD, src_r, dst_r)
    hn2 = _tc_mid(agg1, dsum, bc1.reshape(1, D), Wc2)
    agg2 = _sc_edge_agg(hn2, zerosD, src_r, dst_r)
    hn3 = _tc_mid(agg2, dsum, bc2.reshape(1, D), Wc3)
    agg3 = _sc_edge_agg(hn3, zerosD, src_r, dst_r)

    wl2_pad = jnp.concatenate([Wl2, jnp.zeros((7, D), jnp.float32)], axis=0)
    bl2_pad = jnp.concatenate([bl2, jnp.zeros((7,), jnp.float32)]).reshape(1, 8)
    o = _tc_out(agg3, dsum, bc3.reshape(1, D), batch.reshape(N, 1),
                Wl1, bl1.reshape(1, D), wl2_pad, bl2_pad)
    return o[:, 0]


# speedup vs baseline: 25.2644x; 1.0043x over previous
import functools

import jax
import jax.numpy as jnp
from jax import lax
from jax.experimental import pallas as pl
from jax.experimental.pallas import tpu as pltpu
from jax.experimental.pallas import tpu_sc as plsc

N = 10000
E = 320000
D = 128
G = 64

NC = 2
NS = 16
NW = NC * NS
C = 128
EPT = -(-E // NW)
NPHASE = 2
NCHUNK = 4 * (-(-EPT // (4 * C)))
CH_P = NCHUNK // NPHASE
NHALF_P = CH_P // 2
EPT_PAD = NCHUNK * C
E_PAD = EPT_PAD * NW
NDUMMY = 112
NPAD = N + NDUMMY
ROWS_PT = NPAD // NS

_mesh = plsc.VectorSubcoreMesh(core_axis_name="c", subcore_axis_name="s")


@functools.partial(
    pl.kernel,
    out_type=jax.ShapeDtypeStruct((NC, NPAD, D), jnp.float32),
    mesh=_mesh,
    scratch_types=[
        pltpu.VMEM((NCHUNK, C), jnp.int32),
        pltpu.VMEM((C, D), jnp.float32),
        pltpu.VMEM_SHARED((NPAD, D), jnp.float32),
    ],
)
def _sc_degree(dst_hbm, ones_hbm, zeros_hbm, out_hbm, dst_v, ones_v, deg_sh):
    cid = lax.axis_index("c")
    sid = lax.axis_index("s")
    tid = cid * NS + sid
    r0 = sid * ROWS_PT
    pltpu.sync_copy(zeros_hbm.at[pl.ds(r0, ROWS_PT)], deg_sh.at[pl.ds(r0, ROWS_PT)])
    pltpu.sync_copy(ones_hbm, ones_v)
    pltpu.sync_copy(dst_hbm.at[tid], dst_v)
    plsc.subcore_barrier()

    def body(j, carry):
        pltpu.sync_copy(ones_v, deg_sh.at[dst_v.at[j]], add=True)
        return carry

    lax.fori_loop(0, NCHUNK, body, 0)
    plsc.subcore_barrier()
    pltpu.sync_copy(deg_sh.at[pl.ds(r0, ROWS_PT)], out_hbm.at[cid, pl.ds(r0, ROWS_PT)])


@functools.partial(
    pl.kernel,
    out_type=jax.ShapeDtypeStruct((NC, NPAD, D), jnp.float32),
    mesh=_mesh,
    scratch_types=[
        pltpu.VMEM((CH_P, C), jnp.int32),
        pltpu.VMEM((CH_P, C), jnp.int32),
        pltpu.VMEM((C, D), jnp.float32),
        pltpu.VMEM((C, D), jnp.float32),
        pltpu.VMEM_SHARED((NPAD, D), jnp.float32),
        pltpu.SemaphoreType.DMA,
        pltpu.SemaphoreType.DMA,
        pltpu.SemaphoreType.DMA,
        pltpu.SemaphoreType.DMA,
    ],
)
def _sc_edge_agg(hn_hbm, zeros_hbm, src_hbm, dst_hbm, out_hbm,
                 src_v, dst_v, rows_a, rows_b, acc_sh,
                 sem_a, sem_b, ssem_a, ssem_b):
    cid = lax.axis_index("c")
    sid = lax.axis_index("s")
    tid = cid * NS + sid
    r0 = sid * ROWS_PT

    @pl.when(cid == 0)
    def _():
        pltpu.sync_copy(hn_hbm.at[pl.ds(r0, ROWS_PT)], acc_sh.at[pl.ds(r0, ROWS_PT)])

    @pl.when(cid != 0)
    def _():
        pltpu.sync_copy(zeros_hbm.at[pl.ds(r0, ROWS_PT)], acc_sh.at[pl.ds(r0, ROWS_PT)])

    plsc.subcore_barrier()

    def body(i, carry):
        j0 = 2 * i
        j1 = j0 + 1
        pltpu.async_copy(hn_hbm.at[src_v.at[j1]], rows_b, sem_b)
        pltpu.make_async_copy(hn_hbm.at[src_v.at[j0]], rows_a, sem_a).wait()
        pltpu.sync_copy(rows_a, acc_sh.at[dst_v.at[j0]], add=True)

        @pl.when(i < NHALF_P - 1)
        def _():
            pltpu.async_copy(hn_hbm.at[src_v.at[j0 + 2]], rows_a, sem_a)

        pltpu.make_async_copy(hn_hbm.at[src_v.at[j1]], rows_b, sem_b).wait()
        pltpu.sync_copy(rows_b, acc_sh.at[dst_v.at[j1]], add=True)
        return carry

    for p in range(NPHASE):
        pltpu.sync_copy(src_hbm.at[tid, pl.ds(p * CH_P, CH_P)], src_v)
        pltpu.sync_copy(dst_hbm.at[tid, pl.ds(p * CH_P, CH_P)], dst_v)
        pltpu.async_copy(hn_hbm.at[src_v.at[0]], rows_a, sem_a)
        lax.fori_loop(0, NHALF_P, body, 0)
    plsc.subcore_barrier()
    pltpu.sync_copy(acc_sh.at[pl.ds(r0, ROWS_PT)], out_hbm.at[cid, pl.ds(r0, ROWS_PT)])


def _dis8(dsum):
    deg = dsum[0, :, 0:8] + dsum[1, :, 0:8] + 1.0
    return lax.rsqrt(jnp.maximum(deg, 1e-12))


def _tc_h0_body(x_ref, w0_ref, b0_ref, out_ref):
    out_ref[...] = jnp.maximum(
        lax.dot_general(x_ref[...], w0_ref[...], (((1,), (1,)), ((), ())))
        + b0_ref[...], 0.0)


_tc_h0 = pl.pallas_call(
    _tc_h0_body,
    out_shape=jax.ShapeDtypeStruct((N, D), jnp.float32),
)


def _tc_in_body(h0_ref, wc1_ref, dsum_ref, out_ref, dis_ref):
    dis8 = _dis8(dsum_ref[...])
    dis_ref[...] = dis8
    t = lax.dot_general(h0_ref[...], wc1_ref[...], (((1,), (1,)), ((), ())))
    out_ref[0:N, :] = t * dis8[:N, 0:1]
    out_ref[N:NPAD, :] = jnp.zeros((NDUMMY, D), jnp.float32)


_tc_in = pl.pallas_call(
    _tc_in_body,
    out_shape=[jax.ShapeDtypeStruct((NPAD, D), jnp.float32),
               jax.ShapeDtypeStruct((NPAD, 8), jnp.float32)],
)


def _tc_mid_body(agg_ref, dis_ref, b_ref, w_ref, out_ref):
    disc = dis_ref[0:N, 0:1]
    s = agg_ref[0, 0:N, :] + agg_ref[1, 0:N, :]
    y = jnp.maximum(s * disc + b_ref[...], 0.0)
    t = lax.dot_general(y, w_ref[...], (((1,), (1,)), ((), ())))
    out_ref[0:N, :] = t * disc
    out_ref[N:NPAD, :] = jnp.zeros((NDUMMY, D), jnp.float32)


_tc_mid = pl.pallas_call(
    _tc_mid_body,
    out_shape=jax.ShapeDtypeStruct((NPAD, D), jnp.float32),
)


def _tc_out_body(agg_ref, dis_ref, b_ref, batch_ref, wl1_ref, bl1_ref,
                 wl2_ref, bl2_ref, out_ref):
    disc = dis_ref[0:N, 0:1]
    s = agg_ref[0, 0:N, :] + agg_ref[1, 0:N, :]
    y = jnp.maximum(s * disc + b_ref[...], 0.0)
    seg = lax.broadcasted_iota(jnp.int32, (1, G), 1)
    onehot = (batch_ref[...] == seg).astype(jnp.float32)
    g = lax.dot_general(onehot, y, (((0,), (0,)), ((), ())))
    g2 = jnp.maximum(
        lax.dot_general(g, wl1_ref[...], (((1,), (1,)), ((), ())))
        + bl1_ref[...], 0.0)
    o = lax.dot_general(g2, wl2_ref[...], (((1,), (1,)), ((), ()))) + bl2_ref[...]
    out_ref[...] = o


_tc_out = pl.pallas_call(
    _tc_out_body,
    out_shape=jax.ShapeDtypeStruct((G, 8), jnp.float32),
)


def kernel(x, edge_index, batch, W0, b0, Wc1, bc1, Wc2, bc2, Wc3, bc3,
           Wl1, bl1, Wl2, bl2):
    src = edge_index[0]
    dst = edge_index[1]
    pad_n = E_PAD - E
    padv = (N + (jnp.arange(pad_n, dtype=jnp.int32) % NDUMMY)).astype(jnp.int32)
    src_r = jnp.concatenate([src, padv]).reshape(NW, NCHUNK, C)
    dst_r = jnp.concatenate([dst, padv]).reshape(NW, NCHUNK, C)
    onesD = jnp.ones((C, D), jnp.float32)
    zerosD = jnp.zeros((NPAD, D), jnp.float32)

    dsum = _sc_degree(dst_r, onesD, zerosD)

    h0 = _tc_h0(x, W0, b0.reshape(1, D))
    hn1, dis8 = _tc_in(h0, Wc1, dsum)
    agg1 = _sc_edge_agg(hn1, zerosD, src_r, dst_r)
    hn2 = _tc_mid(agg1, dis8, bc1.reshape(1, D), Wc2)
    agg2 = _sc_edge_agg(hn2, zerosD, src_r, dst_r)
    hn3 = _tc_mid(agg2, dis8, bc2.reshape(1, D), Wc3)
    agg3 = _sc_edge_agg(hn3, zerosD, src_r, dst_r)

    wl2_pad = jnp.concatenate([Wl2, jnp.zeros((7, D), jnp.float32)], axis=0)
    bl2_pad = jnp.concatenate([bl2, jnp.zeros((7,), jnp.float32)]).reshape(1, 8)
    o = _tc_out(agg3, dis8, bc3.reshape(1, D), batch.reshape(N, 1),
                Wl1, bl1.reshape(1, D), wl2_pad, bl2_pad)
    return o[:, 0]


# speedup vs baseline: 25.4690x; 1.0081x over previous
import functools

import jax
import jax.numpy as jnp
from jax import lax
from jax.experimental import pallas as pl
from jax.experimental.pallas import tpu as pltpu
from jax.experimental.pallas import tpu_sc as plsc

N = 10000
E = 320000
D = 128
G = 64

NC = 2
NS = 16
NW = NC * NS
C = 128
EPT = -(-E // NW)
NPHASE = 2
NCHUNK = 4 * (-(-EPT // (4 * C)))
CH_P = NCHUNK // NPHASE
NHALF_P = CH_P // 2
EPT_PAD = NCHUNK * C
E_PAD = EPT_PAD * NW
NDUMMY = 112
NPAD = N + NDUMMY
ROWS_PT = NPAD // NS

_mesh = plsc.VectorSubcoreMesh(core_axis_name="c", subcore_axis_name="s")


@functools.partial(
    pl.kernel,
    out_type=jax.ShapeDtypeStruct((NC, NPAD, D), jnp.float32),
    mesh=_mesh,
    scratch_types=[
        pltpu.VMEM((NCHUNK, C), jnp.int32),
        pltpu.VMEM((C, D), jnp.float32),
        pltpu.VMEM_SHARED((NPAD, D), jnp.float32),
    ],
)
def _sc_degree(dst_hbm, ones_hbm, zeros_hbm, out_hbm, dst_v, ones_v, deg_sh):
    cid = lax.axis_index("c")
    sid = lax.axis_index("s")
    tid = cid * NS + sid
    r0 = sid * ROWS_PT
    pltpu.sync_copy(zeros_hbm.at[pl.ds(r0, ROWS_PT)], deg_sh.at[pl.ds(r0, ROWS_PT)])
    pltpu.sync_copy(ones_hbm, ones_v)
    pltpu.sync_copy(dst_hbm.at[tid], dst_v)
    plsc.subcore_barrier()

    def body(j, carry):
        pltpu.sync_copy(ones_v, deg_sh.at[dst_v.at[j]], add=True)
        return carry

    lax.fori_loop(0, NCHUNK, body, 0)
    plsc.subcore_barrier()
    pltpu.sync_copy(deg_sh.at[pl.ds(r0, ROWS_PT)], out_hbm.at[cid, pl.ds(r0, ROWS_PT)])


@functools.partial(
    pl.kernel,
    out_type=jax.ShapeDtypeStruct((NC, NPAD, D), jnp.float32),
    mesh=_mesh,
    scratch_types=[
        pltpu.VMEM((CH_P, C), jnp.int32),
        pltpu.VMEM((CH_P, C), jnp.int32),
        pltpu.VMEM((C, D), jnp.float32),
        pltpu.VMEM((C, D), jnp.float32),
        pltpu.VMEM_SHARED((NPAD, D), jnp.float32),
        pltpu.SemaphoreType.DMA,
        pltpu.SemaphoreType.DMA,
        pltpu.SemaphoreType.DMA,
        pltpu.SemaphoreType.DMA,
    ],
)
def _sc_edge_agg(hn_hbm, zeros_hbm, src_hbm, dst_hbm, out_hbm,
                 src_v, dst_v, rows_a, rows_b, acc_sh,
                 sem_a, sem_b, ssem_a, ssem_b):
    cid = lax.axis_index("c")
    sid = lax.axis_index("s")
    tid = cid * NS + sid
    r0 = sid * ROWS_PT

    @pl.when(cid == 0)
    def _():
        pltpu.sync_copy(hn_hbm.at[pl.ds(r0, ROWS_PT)], acc_sh.at[pl.ds(r0, ROWS_PT)])

    @pl.when(cid != 0)
    def _():
        pltpu.sync_copy(zeros_hbm.at[pl.ds(r0, ROWS_PT)], acc_sh.at[pl.ds(r0, ROWS_PT)])

    plsc.subcore_barrier()

    def body(i, carry):
        j0 = 2 * i
        j1 = j0 + 1
        pltpu.async_copy(hn_hbm.at[src_v.at[j1]], rows_b, sem_b)
        pltpu.make_async_copy(hn_hbm.at[src_v.at[j0]], rows_a, sem_a).wait()
        pltpu.sync_copy(rows_a, acc_sh.at[dst_v.at[j0]], add=True)

        @pl.when(i < NHALF_P - 1)
        def _():
            pltpu.async_copy(hn_hbm.at[src_v.at[j0 + 2]], rows_a, sem_a)

        pltpu.make_async_copy(hn_hbm.at[src_v.at[j1]], rows_b, sem_b).wait()
        pltpu.sync_copy(rows_b, acc_sh.at[dst_v.at[j1]], add=True)
        return carry

    for p in range(NPHASE):
        pltpu.sync_copy(src_hbm.at[tid, pl.ds(p * CH_P, CH_P)], src_v)
        pltpu.sync_copy(dst_hbm.at[tid, pl.ds(p * CH_P, CH_P)], dst_v)
        pltpu.async_copy(hn_hbm.at[src_v.at[0]], rows_a, sem_a)
        lax.fori_loop(0, NHALF_P, body, 0)
    plsc.subcore_barrier()
    pltpu.sync_copy(acc_sh.at[pl.ds(r0, ROWS_PT)], out_hbm.at[cid, pl.ds(r0, ROWS_PT)])


def _dis8(dsum):
    deg = dsum[0, :, 0:8] + dsum[1, :, 0:8] + 1.0
    return lax.rsqrt(jnp.maximum(deg, 1e-12))


def _tc_h0_body(x_ref, w0_ref, b0_ref, out_ref):
    out_ref[...] = jnp.maximum(
        lax.dot_general(x_ref[...], w0_ref[...], (((1,), (1,)), ((), ())))
        + b0_ref[...], 0.0)


_tc_h0 = pl.pallas_call(
    _tc_h0_body,
    out_shape=jax.ShapeDtypeStruct((N, D), jnp.float32),
)


def _tc_in_body(x_ref, w0_ref, b0_ref, wc1_ref, dsum_ref, out_ref, dis_ref):
    dis8 = _dis8(dsum_ref[...])
    dis_ref[...] = dis8
    h0 = jnp.maximum(
        lax.dot_general(x_ref[...], w0_ref[...], (((1,), (1,)), ((), ())))
        + b0_ref[...], 0.0)
    t = lax.dot_general(h0, wc1_ref[...], (((1,), (1,)), ((), ())))
    out_ref[0:N, :] = t * dis8[:N, 0:1]
    out_ref[N:NPAD, :] = jnp.zeros((NDUMMY, D), jnp.float32)


_tc_in = pl.pallas_call(
    _tc_in_body,
    out_shape=[jax.ShapeDtypeStruct((NPAD, D), jnp.float32),
               jax.ShapeDtypeStruct((NPAD, 8), jnp.float32)],
)


def _tc_mid_body(agg_ref, dis_ref, b_ref, w_ref, out_ref):
    disc = dis_ref[0:N, 0:1]
    s = agg_ref[0, 0:N, :] + agg_ref[1, 0:N, :]
    y = jnp.maximum(s * disc + b_ref[...], 0.0)
    t = lax.dot_general(y, w_ref[...], (((1,), (1,)), ((), ())))
    out_ref[0:N, :] = t * disc
    out_ref[N:NPAD, :] = jnp.zeros((NDUMMY, D), jnp.float32)


_tc_mid = pl.pallas_call(
    _tc_mid_body,
    out_shape=jax.ShapeDtypeStruct((NPAD, D), jnp.float32),
)


def _tc_out_body(agg_ref, dis_ref, b_ref, batch_ref, wl1_ref, bl1_ref,
                 wl2_ref, bl2_ref, out_ref):
    disc = dis_ref[0:N, 0:1]
    s = agg_ref[0, 0:N, :] + agg_ref[1, 0:N, :]
    y = jnp.maximum(s * disc + b_ref[...], 0.0)
    seg = lax.broadcasted_iota(jnp.int32, (1, G), 1)
    onehot = (batch_ref[...] == seg).astype(jnp.float32)
    g = lax.dot_general(onehot, y, (((0,), (0,)), ((), ())))
    g2 = jnp.maximum(
        lax.dot_general(g, wl1_ref[...], (((1,), (1,)), ((), ())))
        + bl1_ref[...], 0.0)
    o = lax.dot_general(g2, wl2_ref[...], (((1,), (1,)), ((), ()))) + bl2_ref[...]
    out_ref[...] = o


_tc_out = pl.pallas_call(
    _tc_out_body,
    out_shape=jax.ShapeDtypeStruct((G, 8), jnp.float32),
)


def kernel(x, edge_index, batch, W0, b0, Wc1, bc1, Wc2, bc2, Wc3, bc3,
           Wl1, bl1, Wl2, bl2):
    src = edge_index[0]
    dst = edge_index[1]
    pad_n = E_PAD - E
    padv = (N + (jnp.arange(pad_n, dtype=jnp.int32) % NDUMMY)).astype(jnp.int32)
    src_r = jnp.concatenate([src, padv]).reshape(NW, NCHUNK, C)
    dst_r = jnp.concatenate([dst, padv]).reshape(NW, NCHUNK, C)
    onesD = jnp.ones((C, D), jnp.float32)
    zerosD = jnp.zeros((NPAD, D), jnp.float32)

    dsum = _sc_degree(dst_r, onesD, zerosD)

    hn1, dis8 = _tc_in(x, W0, b0.reshape(1, D), Wc1, dsum)
    agg1 = _sc_edge_agg(hn1, zerosD, src_r, dst_r)
    hn2 = _tc_mid(agg1, dis8, bc1.reshape(1, D), Wc2)
    agg2 = _sc_edge_agg(hn2, zerosD, src_r, dst_r)
    hn3 = _tc_mid(agg2, dis8, bc2.reshape(1, D), Wc3)
    agg3 = _sc_edge_agg(hn3, zerosD, src_r, dst_r)

    wl2_pad = jnp.concatenate([Wl2, jnp.zeros((7, D), jnp.float32)], axis=0)
    bl2_pad = jnp.concatenate([bl2, jnp.zeros((7,), jnp.float32)]).reshape(1, 8)
    o = _tc_out(agg3, dis8, bc3.reshape(1, D), batch.reshape(N, 1),
                Wl1, bl1.reshape(1, D), wl2_pad, bl2_pad)
    return o[:, 0]
